# baseline (device time: 50257 ns/iter reference)
import jax
import jax.numpy as jnp
from jax import lax
from jax.experimental import pallas as pl
from jax.experimental.pallas import tpu as pltpu

T = 2048
D = 1024
V_SHARD = 16384
HALF = T // 2
C = 8
R = HALF // C
G = 8


def kernel(ids, E):
    ids1d = ids.astype(jnp.int32)
    ids2d = ids1d.reshape(T, 1)

    my_x = lax.axis_index("x")
    my_y = lax.axis_index("y")

    seg_ids = lax.dynamic_slice(ids1d, (my_y * HALF,), (HALF,))
    idx_loc = seg_ids - my_x * V_SHARD
    ownd = (idx_loc >= 0) & (idx_loc < V_SHARD)
    ch_own = ownd.reshape(C, R)
    ch_idx = jnp.where(ownd, idx_loc, 0).reshape(C, R)
    order = jnp.argsort((~ch_own).astype(jnp.int32), axis=1, stable=True)
    srt_idx = jnp.take_along_axis(ch_idx, order, axis=1)
    ranks = jnp.arange(R)[None, :]
    n_c = ch_own.sum(axis=1)
    valid = ranks < n_c[:, None]
    src_flat = jnp.where(valid, srt_idx, 0).reshape(-1).astype(jnp.int32)
    row_in_gbuf = order + (jnp.arange(C) * R)[:, None]
    dst_flat = jnp.where(valid, row_in_gbuf, HALF).reshape(-1).astype(jnp.int32)
    n8 = ((n_c + G - 1) // G).astype(jnp.int32)

    def body(
        src_smem,
        dst_smem,
        n8_smem,
        ids_vmem,
        E_hbm,
        out_ref,
        gbuf_ref,
        part_ref,
        xrecv_ref,
        gsems,
        x_send_sems,
        x_recv_sems,
        y_send_sems,
        y_recv_sems,
    ):
        x = lax.axis_index("x")
        y = lax.axis_index("y")
        xnbr = (1 - x, y)
        ynbr = (x, 1 - y)

        barrier = pltpu.get_barrier_semaphore()
        for nbr in (xnbr, ynbr):
            pl.semaphore_signal(
                barrier, inc=1, device_id=nbr, device_id_type=pl.DeviceIdType.MESH
            )
        pl.semaphore_wait(barrier, 2)

        tok0 = y * HALF

        x_rdmas = []
        y_rdmas = []
        for c in range(C):
            rows = pl.ds(c * R, R)
            tok_rows = pl.ds(tok0 + c * R, R)
            x_rdmas.append(
                pltpu.make_async_remote_copy(
                    src_ref=part_ref.at[rows],
                    dst_ref=xrecv_ref.at[rows],
                    send_sem=x_send_sems.at[c],
                    recv_sem=x_recv_sems.at[c],
                    device_id=xnbr,
                    device_id_type=pl.DeviceIdType.MESH,
                )
            )
            y_rdmas.append(
                pltpu.make_async_remote_copy(
                    src_ref=out_ref.at[tok_rows],
                    dst_ref=out_ref.at[tok_rows],
                    send_sem=y_send_sems.at[c],
                    recv_sem=y_recv_sems.at[c],
                    device_id=ynbr,
                    device_id_type=pl.DeviceIdType.MESH,
                )
            )

        def issue_chunk(c):
            base = c * R

            def body8(k, _):
                off = base + k * G
                for u in range(G):
                    pltpu.make_async_copy(
                        E_hbm.at[pl.ds(src_smem[off + u], 1), :],
                        gbuf_ref.at[pl.ds(dst_smem[off + u], 1), :],
                        gsems.at[c],
                    ).start()
                return 0

            lax.fori_loop(0, n8_smem[c], body8, 0)

        def flush_chunk(c):
            def wait8(k, _):
                for _u in range(G):
                    pltpu.make_async_copy(
                        E_hbm.at[pl.ds(0, 1), :],
                        gbuf_ref.at[pl.ds(0, 1), :],
                        gsems.at[c],
                    ).wait()
                return 0

            lax.fori_loop(0, n8_smem[c], wait8, 0)
            rows = pl.ds(c * R, R)
            part_ref[rows] = gbuf_ref[rows].astype(jnp.bfloat16)
            x_rdmas[c].start()

        issue_chunk(0)
        for c in range(1, C):
            issue_chunk(c)
            flush_chunk(c - 1)
        flush_chunk(C - 1)

        for c in range(C):
            x_rdmas[c].wait_recv()
            rows = pl.ds(c * R, R)
            tok_rows = pl.ds(tok0 + c * R, R)
            mine = (ids_vmem[tok_rows] // V_SHARD) == x
            out_ref[tok_rows] = jnp.where(mine, part_ref[rows], xrecv_ref[rows])
            y_rdmas[c].start()

        for c in range(C):
            y_rdmas[c].wait_recv()

        for c in range(C):
            x_rdmas[c].wait_send()
            y_rdmas[c].wait_send()

    return pl.pallas_call(
        body,
        out_shape=jax.ShapeDtypeStruct((T, D), jnp.bfloat16),
        in_specs=[
            pl.BlockSpec(memory_space=pltpu.SMEM),
            pl.BlockSpec(memory_space=pltpu.SMEM),
            pl.BlockSpec(memory_space=pltpu.SMEM),
            pl.BlockSpec(memory_space=pltpu.VMEM),
            pl.BlockSpec(memory_space=pltpu.HBM),
        ],
        out_specs=pl.BlockSpec(memory_space=pltpu.VMEM),
        scratch_shapes=[
            pltpu.VMEM((HALF + 1, D), jnp.float32),
            pltpu.VMEM((HALF, D), jnp.bfloat16),
            pltpu.VMEM((HALF, D), jnp.bfloat16),
            pltpu.SemaphoreType.DMA((C,)),
            pltpu.SemaphoreType.DMA((C,)),
            pltpu.SemaphoreType.DMA((C,)),
            pltpu.SemaphoreType.DMA((C,)),
            pltpu.SemaphoreType.DMA((C,)),
        ],
        compiler_params=pltpu.CompilerParams(collective_id=0),
    )(src_flat, dst_flat, n8, ids2d, E)
